# Initial kernel scaffold; baseline (speedup 1.0000x reference)
import jax
import jax.numpy as jnp
from jax.experimental import pallas as pl


def kernel(x, W1, b1, ln1_g, ln1_b, memory_slots, positional_encoding, curvature, curv_alpha, W2, b2, W3, b3, W4, b4, ln2_g, ln2_b, temperature):
    raise NotImplementedError("write your pallas kernel here")


# speedup vs baseline: 1.5942x; 1.5942x over previous
import functools

import jax
import jax.numpy as jnp
from jax import lax
from jax.experimental import pallas as pl
from jax.experimental.pallas import tpu as pltpu

D = 16
M = 16384
H = 64
K_SMALL = 32
K_BIG = 48
F = D * 3
INPUT_DIM = 256
ODE_STEPS = 2
ODE_DT = 0.5
LB_MOMENTUM = 0.99
LB_DROP = 0.7
LB_TOP1_AVG0 = 1.0

TOK_TILE = 256
SLOT_TILE = 2048


def _layer_norm(x, g, b, eps=1e-5):
    mu = jnp.mean(x, axis=-1, keepdims=True)
    var = jnp.mean((x - mu) ** 2, axis=-1, keepdims=True)
    return (x - mu) * lax.rsqrt(var + eps) * g + b


def _gelu_exact(x):
    return x * 0.5 * (1.0 + lax.erf(x * (2.0 ** -0.5)))


def _q_kernel(x_ref, w1_ref, b1_ref, g1_ref, bb1_ref, w2_ref, b2_ref,
              w3_ref, b3_ref, q_ref):
    x = x_ref[...]
    man = jnp.dot(x, w1_ref[...].T, preferred_element_type=jnp.float32) + b1_ref[...]
    man = _gelu_exact(_layer_norm(man, g1_ref[...], bb1_ref[...]))
    ev = man
    for _ in range(ODE_STEPS):
        h = jnp.tanh(jnp.dot(ev, w2_ref[...].T, preferred_element_type=jnp.float32) + b2_ref[...])
        dx = jnp.dot(h, w3_ref[...].T, preferred_element_type=jnp.float32) + b3_ref[...]
        ev = ev + ODE_DT * dx
    q_ref[...] = ev


def _dist_kernel(q_ref, mp_ref, curv_ref, invt_ref, alpha_ref, dist_ref):
    q = q_ref[...]
    mp = mp_ref[...]
    curv = curv_ref[...]
    inv_t = invt_ref[0, 0]
    alpha = alpha_ref[0, 0]
    q2 = jnp.sum(q * q, axis=-1, keepdims=True)
    m2 = jnp.sum(mp * mp, axis=-1)
    cross = jnp.dot(q, mp.T, preferred_element_type=jnp.float32)
    d2 = q2 + m2[None, :] - 2.0 * cross
    dist = jnp.sqrt(jnp.clip(d2, 0.0, None)) * inv_t
    cw = jnp.exp(-alpha * jnp.sqrt(jnp.sum(curv * curv, axis=-1)))
    dist_ref[...] = dist * cw[None, :]


def _out_kernel(att_ref, w4_ref, b4_ref, g2_ref, bb2_ref, out_ref):
    att = att_ref[...]
    y = jnp.dot(att, w4_ref[...].T, preferred_element_type=jnp.float32) + b4_ref[...]
    out_ref[...] = _gelu_exact(_layer_norm(y, g2_ref[...], bb2_ref[...]))


def kernel(x, W1, b1, ln1_g, ln1_b, memory_slots, positional_encoding, curvature,
           curv_alpha, W2, b2, W3, b3, W4, b4, ln2_g, ln2_b, temperature):
    B, S, _ = x.shape
    N = B * S
    xf = x.reshape(N, INPUT_DIM)
    n_tok = N // TOK_TILE
    n_slot = M // SLOT_TILE

    q = pl.pallas_call(
        _q_kernel,
        grid=(n_tok,),
        in_specs=[
            pl.BlockSpec((TOK_TILE, INPUT_DIM), lambda i: (i, 0)),
            pl.BlockSpec((F, INPUT_DIM), lambda i: (0, 0)),
            pl.BlockSpec((1, F), lambda i: (0, 0)),
            pl.BlockSpec((1, F), lambda i: (0, 0)),
            pl.BlockSpec((1, F), lambda i: (0, 0)),
            pl.BlockSpec((128, F), lambda i: (0, 0)),
            pl.BlockSpec((1, 128), lambda i: (0, 0)),
            pl.BlockSpec((F, 128), lambda i: (0, 0)),
            pl.BlockSpec((1, F), lambda i: (0, 0)),
        ],
        out_specs=pl.BlockSpec((TOK_TILE, F), lambda i: (i, 0)),
        out_shape=jax.ShapeDtypeStruct((N, F), jnp.float32),
    )(xf, W1, b1.reshape(1, F), ln1_g.reshape(1, F), ln1_b.reshape(1, F),
      W2, b2.reshape(1, 128), W3, b3.reshape(1, F))

    mem_pos = positional_encoding.reshape(M, F)
    inv_t = (1.0 / jnp.maximum(temperature, 1e-6)).reshape(1, 1).astype(jnp.float32)

    dist = pl.pallas_call(
        _dist_kernel,
        grid=(n_tok, n_slot),
        in_specs=[
            pl.BlockSpec((TOK_TILE, F), lambda i, j: (i, 0)),
            pl.BlockSpec((SLOT_TILE, F), lambda i, j: (j, 0)),
            pl.BlockSpec((SLOT_TILE, D), lambda i, j: (j, 0)),
            pl.BlockSpec((1, 1), lambda i, j: (0, 0), memory_space=pltpu.SMEM),
            pl.BlockSpec((1, 1), lambda i, j: (0, 0), memory_space=pltpu.SMEM),
        ],
        out_specs=pl.BlockSpec((TOK_TILE, SLOT_TILE), lambda i, j: (i, j)),
        out_shape=jax.ShapeDtypeStruct((N, M), jnp.float32),
    )(q, mem_pos, curvature, inv_t,
      curv_alpha.reshape(1, 1).astype(jnp.float32))

    neg_dtop, itop = lax.top_k(-dist, K_BIG)
    dtop = -neg_dtop

    top1 = jnp.mean(dtop[:, 0])
    lb_new = LB_MOMENTUM * LB_TOP1_AVG0 + (1.0 - LB_MOMENTUM) * top1
    fire = top1 < LB_DROP * lb_new

    mem = memory_slots[itop]
    w_big = jax.nn.softmax(-dtop, axis=-1)
    w_small = jax.nn.softmax(-dtop[:, :K_SMALL], axis=-1)
    att_big = jnp.einsum('nk,nkh->nh', w_big, mem)
    att_small = jnp.einsum('nk,nkh->nh', w_small, mem[:, :K_SMALL])
    attended = jnp.where(fire, att_big, att_small)

    out = pl.pallas_call(
        _out_kernel,
        grid=(n_tok,),
        in_specs=[
            pl.BlockSpec((TOK_TILE, H), lambda i: (i, 0)),
            pl.BlockSpec((INPUT_DIM, H), lambda i: (0, 0)),
            pl.BlockSpec((1, INPUT_DIM), lambda i: (0, 0)),
            pl.BlockSpec((1, INPUT_DIM), lambda i: (0, 0)),
            pl.BlockSpec((1, INPUT_DIM), lambda i: (0, 0)),
        ],
        out_specs=pl.BlockSpec((TOK_TILE, INPUT_DIM), lambda i: (i, 0)),
        out_shape=jax.ShapeDtypeStruct((N, INPUT_DIM), jnp.float32),
    )(attended, W4, b4.reshape(1, INPUT_DIM), ln2_g.reshape(1, INPUT_DIM),
      ln2_b.reshape(1, INPUT_DIM))

    return out.reshape(B, S, INPUT_DIM)


# speedup vs baseline: 4.7891x; 3.0041x over previous
import functools

import jax
import jax.numpy as jnp
from jax import lax
from jax.experimental import pallas as pl
from jax.experimental.pallas import tpu as pltpu
from jax.experimental.pallas import tpu_sc as plsc

D = 16
M = 16384
H = 64
K_SMALL = 32
K_BIG = 48
F = D * 3
INPUT_DIM = 256
ODE_STEPS = 2
ODE_DT = 0.5
LB_MOMENTUM = 0.99
LB_DROP = 0.7
LB_TOP1_AVG0 = 1.0

TOK_TILE = 256
SLOT_TILE = 2048
BLK = 32
NBLK = M // BLK
NC, NS, LANES = 2, 16, 16
NW = NC * NS
N_TOKENS = 2048
TOK_PER_W = N_TOKENS // NW
INF_BITS = 0x7F800000


def _layer_norm(x, g, b, eps=1e-5):
    mu = jnp.mean(x, axis=-1, keepdims=True)
    var = jnp.mean((x - mu) ** 2, axis=-1, keepdims=True)
    return (x - mu) * lax.rsqrt(var + eps) * g + b


def _gelu_exact(x):
    return x * 0.5 * (1.0 + lax.erf(x * (2.0 ** -0.5)))


def _q_kernel(x_ref, w1_ref, b1_ref, g1_ref, bb1_ref, w2_ref, b2_ref,
              w3_ref, b3_ref, q_ref):
    x = x_ref[...]
    man = jnp.dot(x, w1_ref[...].T, preferred_element_type=jnp.float32) + b1_ref[...]
    man = _gelu_exact(_layer_norm(man, g1_ref[...], bb1_ref[...]))
    ev = man
    for _ in range(ODE_STEPS):
        h = jnp.tanh(jnp.dot(ev, w2_ref[...].T, preferred_element_type=jnp.float32) + b2_ref[...])
        dx = jnp.dot(h, w3_ref[...].T, preferred_element_type=jnp.float32) + b3_ref[...]
        ev = ev + ODE_DT * dx
    q_ref[...] = ev


def _dist_kernel(q_ref, mp_ref, curv_ref, invt_ref, alpha_ref,
                 dist_ref, bm_ref, d0_ref):
    j = pl.program_id(1)
    q = q_ref[...]
    mp = mp_ref[...]
    curv = curv_ref[...]
    inv_t = invt_ref[0, 0]
    alpha = alpha_ref[0, 0]
    q2 = jnp.sum(q * q, axis=-1, keepdims=True)
    m2 = jnp.sum(mp * mp, axis=-1)
    cross = jnp.dot(q, mp.T, preferred_element_type=jnp.float32)
    d2 = q2 + m2[None, :] - 2.0 * cross
    dist = jnp.sqrt(jnp.clip(d2, 0.0, None)) * inv_t
    cw = jnp.exp(-alpha * jnp.sqrt(jnp.sum(curv * curv, axis=-1)))
    dist = dist * cw[None, :]
    dist_ref[...] = dist
    bm_ref[0] = jnp.min(dist.reshape(TOK_TILE, SLOT_TILE // BLK, BLK), axis=-1)
    rowmin = jnp.min(dist, axis=-1, keepdims=True)

    @pl.when(j == 0)
    def _():
        d0_ref[...] = rowmin

    @pl.when(j > 0)
    def _():
        d0_ref[...] = jnp.minimum(d0_ref[...], rowmin)


def _attend_kernel(dist_ref, thr_ref, d0_ref, d0row_ref, mem_ref,
                   w4_ref, b4_ref, g2_ref, bb2_ref, out_ref,
                   acc_ref, ssum_ref):
    j = pl.program_id(1)
    nj = pl.num_programs(1)

    @pl.when(j == 0)
    def _():
        acc_ref[...] = jnp.zeros_like(acc_ref)
        ssum_ref[...] = jnp.zeros_like(ssum_ref)

    top1 = jnp.mean(d0row_ref[...])
    lb_new = LB_MOMENTUM * LB_TOP1_AVG0 + (1.0 - LB_MOMENTUM) * top1
    fire = top1 < LB_DROP * lb_new
    thr = jnp.where(fire, thr_ref[:, 0:1], thr_ref[:, 1:2])
    d0 = d0_ref[...]
    dist = dist_ref[...]
    w = jnp.where(dist <= thr, jnp.exp(d0 - dist), 0.0)
    acc_ref[...] += jnp.dot(w, mem_ref[...], preferred_element_type=jnp.float32)
    ssum_ref[...] += jnp.sum(w, axis=-1, keepdims=True)

    @pl.when(j == nj - 1)
    def _():
        att = acc_ref[...] / ssum_ref[...]
        y = jnp.dot(att, w4_ref[...].T, preferred_element_type=jnp.float32) + b4_ref[...]
        out_ref[...] = _gelu_exact(_layer_norm(y, g2_ref[...], bb2_ref[...]))


def _sc_body(dist_hbm, bm_hbm, thr_hbm,
             bm_v, cand_v, shf_v, shi_v, thr_v, sm, sem):
    wid = lax.axis_index("s") * NC + lax.axis_index("c")
    t0 = wid * TOK_PER_W
    pltpu.sync_copy(bm_hbm.at[pl.ds(t0, TOK_PER_W)], bm_v)

    lane = lax.iota(jnp.int32, LANES)
    ones = jnp.ones((LANES,), jnp.int32)
    zeros = jnp.zeros((LANES,), jnp.int32)

    def lred_i_sum(v):
        shi_v[pl.ds(LANES, LANES)] = zeros
        cur = v
        for off in (8, 4, 2, 1):
            shi_v[pl.ds(0, LANES)] = cur
            cur = cur + shi_v[pl.ds(off, LANES)]
        return cur[0]

    def bsearch(read_chunk, n_chunks, k):
        def it(_, lh):
            lo, hi = lh
            mid = lo + (hi - lo) // 2
            thr = jnp.broadcast_to(
                lax.bitcast_convert_type(mid, jnp.float32), (LANES,))
            def cb(c, acc):
                return acc + jnp.where(read_chunk(c) <= thr, ones, zeros)
            cnt = lred_i_sum(lax.fori_loop(0, n_chunks, cb, zeros))
            ge = cnt >= k
            return jnp.where(ge, lo, mid + 1), jnp.where(ge, mid, hi)
        _, hi = lax.fori_loop(0, 31, it, (jnp.int32(0), jnp.int32(INF_BITS)))
        return jnp.broadcast_to(
            lax.bitcast_convert_type(hi, jnp.float32), (LANES,))

    def token_body(ti, _):
        t = t0 + ti

        def rd_bm(c):
            return bm_v[ti, pl.ds(c * LANES, LANES)]
        tau0 = bsearch(rd_bm, NBLK // LANES, K_BIG)
        tau0_s = tau0[0]

        def cb1(c, cur):
            v = bm_v[ti, pl.ds(c * LANES, LANES)]
            for i in range(LANES):
                sm[cur] = c * LANES + i
                sel = (v[i] <= tau0_s).astype(jnp.int32)
                cur = jnp.minimum(cur + sel, K_BIG)
            return cur
        lax.fori_loop(0, NBLK // LANES, cb1, jnp.int32(0))

        def fire_b(jj, c):
            base = t * M + sm[jj] * BLK
            pltpu.async_copy(dist_hbm.at[pl.ds(base, BLK)],
                             cand_v.at[pl.ds(jj * BLK, BLK)], sem)
            return c + 1
        lax.fori_loop(0, K_BIG, fire_b, jnp.int32(0))

        def drain_b(jj, c):
            pltpu.make_async_copy(dist_hbm.at[pl.ds(0, BLK)],
                                  cand_v.at[pl.ds(0, BLK)], sem).wait()
            return c + 1
        lax.fori_loop(0, K_BIG, drain_b, jnp.int32(0))

        def rd_cand(c):
            return cand_v[pl.ds(c * LANES, LANES)]
        n_chunks = (K_BIG * BLK) // LANES
        v48 = bsearch(rd_cand, n_chunks, K_BIG)
        v32 = bsearch(rd_cand, n_chunks, K_SMALL)
        thr_v[ti, pl.ds(0, LANES)] = jnp.where(lane == 0, v48, v32)
        return 0

    lax.fori_loop(0, TOK_PER_W, token_body, 0)
    pltpu.sync_copy(thr_v, thr_hbm.at[pl.ds(t0, TOK_PER_W)])


def _sc_thresholds(dist, bm):
    mesh = plsc.VectorSubcoreMesh(core_axis_name="c", subcore_axis_name="s",
                                  num_cores=NC, num_subcores=NS)
    f = pl.kernel(
        _sc_body,
        out_type=[jax.ShapeDtypeStruct((N_TOKENS, LANES), jnp.float32)],
        mesh=mesh,
        scratch_types=[
            pltpu.VMEM((TOK_PER_W, NBLK), jnp.float32),
            pltpu.VMEM((K_BIG * BLK,), jnp.float32),
            pltpu.VMEM((2 * LANES,), jnp.float32),
            pltpu.VMEM((2 * LANES,), jnp.int32),
            pltpu.VMEM((TOK_PER_W, LANES), jnp.float32),
            pltpu.SMEM((K_BIG + LANES,), jnp.int32),
            pltpu.SemaphoreType.DMA,
        ],
    )
    return f(dist.reshape(N_TOKENS * M), bm)


def kernel(x, W1, b1, ln1_g, ln1_b, memory_slots, positional_encoding, curvature,
           curv_alpha, W2, b2, W3, b3, W4, b4, ln2_g, ln2_b, temperature):
    B, S, _ = x.shape
    N = B * S
    xf = x.reshape(N, INPUT_DIM)
    n_tok = N // TOK_TILE
    n_slot = M // SLOT_TILE

    q = pl.pallas_call(
        _q_kernel,
        grid=(n_tok,),
        in_specs=[
            pl.BlockSpec((TOK_TILE, INPUT_DIM), lambda i: (i, 0)),
            pl.BlockSpec((F, INPUT_DIM), lambda i: (0, 0)),
            pl.BlockSpec((1, F), lambda i: (0, 0)),
            pl.BlockSpec((1, F), lambda i: (0, 0)),
            pl.BlockSpec((1, F), lambda i: (0, 0)),
            pl.BlockSpec((128, F), lambda i: (0, 0)),
            pl.BlockSpec((1, 128), lambda i: (0, 0)),
            pl.BlockSpec((F, 128), lambda i: (0, 0)),
            pl.BlockSpec((1, F), lambda i: (0, 0)),
        ],
        out_specs=pl.BlockSpec((TOK_TILE, F), lambda i: (i, 0)),
        out_shape=jax.ShapeDtypeStruct((N, F), jnp.float32),
    )(xf, W1, b1.reshape(1, F), ln1_g.reshape(1, F), ln1_b.reshape(1, F),
      W2, b2.reshape(1, 128), W3, b3.reshape(1, F))

    mem_pos = positional_encoding.reshape(M, F)
    inv_t = (1.0 / jnp.maximum(temperature, 1e-6)).reshape(1, 1).astype(jnp.float32)

    dist, bm3, d0 = pl.pallas_call(
        _dist_kernel,
        grid=(n_tok, n_slot),
        in_specs=[
            pl.BlockSpec((TOK_TILE, F), lambda i, j: (i, 0)),
            pl.BlockSpec((SLOT_TILE, F), lambda i, j: (j, 0)),
            pl.BlockSpec((SLOT_TILE, D), lambda i, j: (j, 0)),
            pl.BlockSpec((1, 1), lambda i, j: (0, 0), memory_space=pltpu.SMEM),
            pl.BlockSpec((1, 1), lambda i, j: (0, 0), memory_space=pltpu.SMEM),
        ],
        out_specs=[
            pl.BlockSpec((TOK_TILE, SLOT_TILE), lambda i, j: (i, j)),
            pl.BlockSpec((1, TOK_TILE, SLOT_TILE // BLK), lambda i, j: (j, i, 0)),
            pl.BlockSpec((TOK_TILE, 1), lambda i, j: (i, 0)),
        ],
        out_shape=[
            jax.ShapeDtypeStruct((N, M), jnp.float32),
            jax.ShapeDtypeStruct((n_slot, N, SLOT_TILE // BLK), jnp.float32),
            jax.ShapeDtypeStruct((N, 1), jnp.float32),
        ],
    )(q, mem_pos, curvature, inv_t,
      curv_alpha.reshape(1, 1).astype(jnp.float32))

    bm = bm3.transpose(1, 0, 2).reshape(N, NBLK)
    thr = _sc_thresholds(dist, bm)[0]

    out = pl.pallas_call(
        _attend_kernel,
        grid=(n_tok, n_slot),
        in_specs=[
            pl.BlockSpec((TOK_TILE, SLOT_TILE), lambda i, j: (i, j)),
            pl.BlockSpec((TOK_TILE, LANES), lambda i, j: (i, 0)),
            pl.BlockSpec((TOK_TILE, 1), lambda i, j: (i, 0)),
            pl.BlockSpec((1, N), lambda i, j: (0, 0)),
            pl.BlockSpec((SLOT_TILE, H), lambda i, j: (j, 0)),
            pl.BlockSpec((INPUT_DIM, H), lambda i, j: (0, 0)),
            pl.BlockSpec((1, INPUT_DIM), lambda i, j: (0, 0)),
            pl.BlockSpec((1, INPUT_DIM), lambda i, j: (0, 0)),
            pl.BlockSpec((1, INPUT_DIM), lambda i, j: (0, 0)),
        ],
        out_specs=pl.BlockSpec((TOK_TILE, INPUT_DIM), lambda i, j: (i, 0)),
        out_shape=jax.ShapeDtypeStruct((N, INPUT_DIM), jnp.float32),
        scratch_shapes=[
            pltpu.VMEM((TOK_TILE, H), jnp.float32),
            pltpu.VMEM((TOK_TILE, 1), jnp.float32),
        ],
    )(dist, thr, d0, d0.reshape(1, N), memory_slots, W4,
      b4.reshape(1, INPUT_DIM), ln2_g.reshape(1, INPUT_DIM),
      ln2_b.reshape(1, INPUT_DIM))

    return out.reshape(B, S, INPUT_DIM)


# speedup vs baseline: 12.8334x; 2.6797x over previous
import functools

import jax
import jax.numpy as jnp
from jax import lax
from jax.experimental import pallas as pl
from jax.experimental.pallas import tpu as pltpu
from jax.experimental.pallas import tpu_sc as plsc

D = 16
M = 16384
H = 64
K_SMALL = 32
K_BIG = 48
F = D * 3
INPUT_DIM = 256
ODE_STEPS = 2
ODE_DT = 0.5
LB_MOMENTUM = 0.99
LB_DROP = 0.7
LB_TOP1_AVG0 = 1.0

TOK_TILE = 256
SLOT_TILE = 2048
BLK = 32
NBLK = M // BLK
NC, NS, LANES = 2, 16, 16
NW = NC * NS
N_TOKENS = 2048
TOK_PER_W = N_TOKENS // NW
INF_BITS = 0x7F800000


def _layer_norm(x, g, b, eps=1e-5):
    mu = jnp.mean(x, axis=-1, keepdims=True)
    var = jnp.mean((x - mu) ** 2, axis=-1, keepdims=True)
    return (x - mu) * lax.rsqrt(var + eps) * g + b


def _gelu_exact(x):
    return x * 0.5 * (1.0 + lax.erf(x * (2.0 ** -0.5)))


def _q_kernel(x_ref, w1_ref, b1_ref, g1_ref, bb1_ref, w2_ref, b2_ref,
              w3_ref, b3_ref, q_ref):
    x = x_ref[...]
    man = jnp.dot(x, w1_ref[...].T, preferred_element_type=jnp.float32) + b1_ref[...]
    man = _gelu_exact(_layer_norm(man, g1_ref[...], bb1_ref[...]))
    ev = man
    for _ in range(ODE_STEPS):
        h = jnp.tanh(jnp.dot(ev, w2_ref[...].T, preferred_element_type=jnp.float32) + b2_ref[...])
        dx = jnp.dot(h, w3_ref[...].T, preferred_element_type=jnp.float32) + b3_ref[...]
        ev = ev + ODE_DT * dx
    q_ref[...] = ev


def _dist_kernel(q_ref, mp_ref, curv_ref, invt_ref, alpha_ref,
                 dist_ref, bm_ref, d0_ref):
    j = pl.program_id(1)
    q = q_ref[...]
    mp = mp_ref[...]
    curv = curv_ref[...]
    inv_t = invt_ref[0, 0]
    alpha = alpha_ref[0, 0]
    q2 = jnp.sum(q * q, axis=-1, keepdims=True)
    m2 = jnp.sum(mp * mp, axis=-1)
    cross = jnp.dot(q, mp.T, preferred_element_type=jnp.float32)
    d2 = q2 + m2[None, :] - 2.0 * cross
    dist = jnp.sqrt(jnp.clip(d2, 0.0, None)) * inv_t
    cw = jnp.exp(-alpha * jnp.sqrt(jnp.sum(curv * curv, axis=-1)))
    dist = dist * cw[None, :]
    dist_ref[...] = dist
    bm_ref[0] = jnp.min(dist.reshape(TOK_TILE, SLOT_TILE // BLK, BLK), axis=-1)
    rowmin = jnp.min(dist, axis=-1, keepdims=True)

    @pl.when(j == 0)
    def _():
        d0_ref[...] = rowmin

    @pl.when(j > 0)
    def _():
        d0_ref[...] = jnp.minimum(d0_ref[...], rowmin)


def _attend_kernel(dist_ref, thr_ref, d0_ref, d0row_ref, mem_ref,
                   w4_ref, b4_ref, g2_ref, bb2_ref, out_ref,
                   acc_ref, ssum_ref):
    j = pl.program_id(1)
    nj = pl.num_programs(1)

    @pl.when(j == 0)
    def _():
        acc_ref[...] = jnp.zeros_like(acc_ref)
        ssum_ref[...] = jnp.zeros_like(ssum_ref)

    top1 = jnp.mean(d0row_ref[...])
    lb_new = LB_MOMENTUM * LB_TOP1_AVG0 + (1.0 - LB_MOMENTUM) * top1
    fire = top1 < LB_DROP * lb_new
    thr = jnp.where(fire, thr_ref[:, 0:1], thr_ref[:, 1:2])
    d0 = d0_ref[...]
    dist = dist_ref[...]
    w = jnp.where(dist <= thr, jnp.exp(d0 - dist), 0.0)
    acc_ref[...] += jnp.dot(w, mem_ref[...], preferred_element_type=jnp.float32)
    ssum_ref[...] += jnp.sum(w, axis=-1, keepdims=True)

    @pl.when(j == nj - 1)
    def _():
        att = acc_ref[...] / ssum_ref[...]
        y = jnp.dot(att, w4_ref[...].T, preferred_element_type=jnp.float32) + b4_ref[...]
        out_ref[...] = _gelu_exact(_layer_norm(y, g2_ref[...], bb2_ref[...]))


def _tau0_kernel(bm3_ref, tau0_ref):
    bm = bm3_ref[...]
    n_tok = bm.shape[1]
    lo0 = jnp.zeros((n_tok, 1), jnp.int32)
    hi0 = jnp.full((n_tok, 1), INF_BITS, jnp.int32)

    def it(_, lh):
        lo, hi = lh
        mid = lo + (hi - lo) // 2
        thr = lax.bitcast_convert_type(mid, jnp.float32)
        cnt = jnp.sum((bm <= thr[None]).astype(jnp.int32), axis=(0, 2),
                      keepdims=True)[0]
        ge = cnt >= K_BIG
        return jnp.where(ge, lo, mid + 1), jnp.where(ge, mid, hi)

    _, hi = lax.fori_loop(0, 31, it, (lo0, hi0))
    tau0_ref[...] = jnp.broadcast_to(
        lax.bitcast_convert_type(hi, jnp.float32), tau0_ref.shape)


def _rank_kernel(cand_ref, thr_ref):
    cand = cand_ref[...]
    n_tok = cand.shape[0]

    def search(k, lo0, hi0):
        def it(_, lh):
            lo, hi = lh
            mid = lo + (hi - lo) // 2
            thr = lax.bitcast_convert_type(mid, jnp.float32)
            cnt = jnp.sum((cand <= thr).astype(jnp.int32), axis=1,
                          keepdims=True)
            ge = cnt >= k
            return jnp.where(ge, lo, mid + 1), jnp.where(ge, mid, hi)
        _, hi = lax.fori_loop(0, 31, it, (lo0, hi0))
        return hi

    lo0 = jnp.zeros((n_tok, 1), jnp.int32)
    hi0 = jnp.full((n_tok, 1), INF_BITS, jnp.int32)
    h48 = search(K_BIG, lo0, hi0)
    h32 = search(K_SMALL, lo0, h48)
    v48 = lax.bitcast_convert_type(h48, jnp.float32)
    v32 = lax.bitcast_convert_type(h32, jnp.float32)
    is0 = lax.broadcasted_iota(jnp.int32, thr_ref.shape, 1) == 0
    thr_ref[...] = jnp.where(is0, v48, v32)


def _sc_body(dist_hbm, bm3_hbm, tau0_hbm, cand_hbm,
             bm_v, cand_v, tau_v, sm, sem):
    wid = lax.axis_index("s") * NC + lax.axis_index("c")
    t0 = wid * TOK_PER_W
    pltpu.sync_copy(bm3_hbm.at[:, pl.ds(t0, TOK_PER_W), :], bm_v)
    pltpu.sync_copy(tau0_hbm.at[pl.ds(t0, TOK_PER_W)], tau_v)

    def token_body(ti, _):
        t = t0 + ti
        tau0_s = tau_v[ti, pl.ds(0, LANES)][0]

        def cb1(c, cur):
            j = c // 4
            kk = (c % 4) * LANES
            v = bm_v[j, ti, pl.ds(kk, LANES)]
            for i in range(LANES):
                sm[cur] = j * (SLOT_TILE // BLK) + kk + i
                sel = (v[i] <= tau0_s).astype(jnp.int32)
                cur = jnp.minimum(cur + sel, K_BIG)
            return cur
        lax.fori_loop(0, NBLK // LANES, cb1, jnp.int32(0))

        def fire_b(jj, c):
            base = t * M + sm[jj] * BLK
            pltpu.async_copy(dist_hbm.at[pl.ds(base, BLK)],
                             cand_v.at[pl.ds(jj * BLK, BLK)], sem)
            return c + 1
        lax.fori_loop(0, K_BIG, fire_b, jnp.int32(0))

        def drain_b(jj, c):
            pltpu.make_async_copy(dist_hbm.at[pl.ds(0, BLK)],
                                  cand_v.at[pl.ds(0, BLK)], sem).wait()
            return c + 1
        lax.fori_loop(0, K_BIG, drain_b, jnp.int32(0))

        pltpu.sync_copy(cand_v, cand_hbm.at[t])
        return 0

    lax.fori_loop(0, TOK_PER_W, token_body, 0)


def _sc_gather_candidates(dist, bm3, tau0):
    mesh = plsc.VectorSubcoreMesh(core_axis_name="c", subcore_axis_name="s",
                                  num_cores=NC, num_subcores=NS)
    f = pl.kernel(
        _sc_body,
        out_type=[jax.ShapeDtypeStruct((N_TOKENS, K_BIG * BLK), jnp.float32)],
        mesh=mesh,
        scratch_types=[
            pltpu.VMEM((M // SLOT_TILE, TOK_PER_W, SLOT_TILE // BLK),
                       jnp.float32),
            pltpu.VMEM((K_BIG * BLK,), jnp.float32),
            pltpu.VMEM((TOK_PER_W, LANES), jnp.float32),
            pltpu.SMEM((K_BIG + LANES,), jnp.int32),
            pltpu.SemaphoreType.DMA,
        ],
    )
    return f(dist.reshape(N_TOKENS * M), bm3, tau0)[0]


def kernel(x, W1, b1, ln1_g, ln1_b, memory_slots, positional_encoding, curvature,
           curv_alpha, W2, b2, W3, b3, W4, b4, ln2_g, ln2_b, temperature):
    B, S, _ = x.shape
    N = B * S
    xf = x.reshape(N, INPUT_DIM)
    n_tok = N // TOK_TILE
    n_slot = M // SLOT_TILE

    q = pl.pallas_call(
        _q_kernel,
        grid=(n_tok,),
        in_specs=[
            pl.BlockSpec((TOK_TILE, INPUT_DIM), lambda i: (i, 0)),
            pl.BlockSpec((F, INPUT_DIM), lambda i: (0, 0)),
            pl.BlockSpec((1, F), lambda i: (0, 0)),
            pl.BlockSpec((1, F), lambda i: (0, 0)),
            pl.BlockSpec((1, F), lambda i: (0, 0)),
            pl.BlockSpec((128, F), lambda i: (0, 0)),
            pl.BlockSpec((1, 128), lambda i: (0, 0)),
            pl.BlockSpec((F, 128), lambda i: (0, 0)),
            pl.BlockSpec((1, F), lambda i: (0, 0)),
        ],
        out_specs=pl.BlockSpec((TOK_TILE, F), lambda i: (i, 0)),
        out_shape=jax.ShapeDtypeStruct((N, F), jnp.float32),
    )(xf, W1, b1.reshape(1, F), ln1_g.reshape(1, F), ln1_b.reshape(1, F),
      W2, b2.reshape(1, 128), W3, b3.reshape(1, F))

    mem_pos = positional_encoding.reshape(M, F)
    inv_t = (1.0 / jnp.maximum(temperature, 1e-6)).reshape(1, 1).astype(jnp.float32)

    dist, bm3, d0 = pl.pallas_call(
        _dist_kernel,
        grid=(n_tok, n_slot),
        in_specs=[
            pl.BlockSpec((TOK_TILE, F), lambda i, j: (i, 0)),
            pl.BlockSpec((SLOT_TILE, F), lambda i, j: (j, 0)),
            pl.BlockSpec((SLOT_TILE, D), lambda i, j: (j, 0)),
            pl.BlockSpec((1, 1), lambda i, j: (0, 0), memory_space=pltpu.SMEM),
            pl.BlockSpec((1, 1), lambda i, j: (0, 0), memory_space=pltpu.SMEM),
        ],
        out_specs=[
            pl.BlockSpec((TOK_TILE, SLOT_TILE), lambda i, j: (i, j)),
            pl.BlockSpec((1, TOK_TILE, SLOT_TILE // BLK), lambda i, j: (j, i, 0)),
            pl.BlockSpec((TOK_TILE, 1), lambda i, j: (i, 0)),
        ],
        out_shape=[
            jax.ShapeDtypeStruct((N, M), jnp.float32),
            jax.ShapeDtypeStruct((n_slot, N, SLOT_TILE // BLK), jnp.float32),
            jax.ShapeDtypeStruct((N, 1), jnp.float32),
        ],
    )(q, mem_pos, curvature, inv_t,
      curv_alpha.reshape(1, 1).astype(jnp.float32))

    tau0 = pl.pallas_call(
        _tau0_kernel,
        grid=(n_tok,),
        in_specs=[pl.BlockSpec((n_slot, TOK_TILE, SLOT_TILE // BLK),
                               lambda i: (0, i, 0))],
        out_specs=pl.BlockSpec((TOK_TILE, LANES), lambda i: (i, 0)),
        out_shape=jax.ShapeDtypeStruct((N, LANES), jnp.float32),
    )(bm3)

    cand = _sc_gather_candidates(dist, bm3, tau0)

    thr = pl.pallas_call(
        _rank_kernel,
        grid=(n_tok,),
        in_specs=[pl.BlockSpec((TOK_TILE, K_BIG * BLK), lambda i: (i, 0))],
        out_specs=pl.BlockSpec((TOK_TILE, LANES), lambda i: (i, 0)),
        out_shape=jax.ShapeDtypeStruct((N, LANES), jnp.float32),
    )(cand)

    out = pl.pallas_call(
        _attend_kernel,
        grid=(n_tok, n_slot),
        in_specs=[
            pl.BlockSpec((TOK_TILE, SLOT_TILE), lambda i, j: (i, j)),
            pl.BlockSpec((TOK_TILE, LANES), lambda i, j: (i, 0)),
            pl.BlockSpec((TOK_TILE, 1), lambda i, j: (i, 0)),
            pl.BlockSpec((1, N), lambda i, j: (0, 0)),
            pl.BlockSpec((SLOT_TILE, H), lambda i, j: (j, 0)),
            pl.BlockSpec((INPUT_DIM, H), lambda i, j: (0, 0)),
            pl.BlockSpec((1, INPUT_DIM), lambda i, j: (0, 0)),
            pl.BlockSpec((1, INPUT_DIM), lambda i, j: (0, 0)),
            pl.BlockSpec((1, INPUT_DIM), lambda i, j: (0, 0)),
        ],
        out_specs=pl.BlockSpec((TOK_TILE, INPUT_DIM), lambda i, j: (i, 0)),
        out_shape=jax.ShapeDtypeStruct((N, INPUT_DIM), jnp.float32),
        scratch_shapes=[
            pltpu.VMEM((TOK_TILE, H), jnp.float32),
            pltpu.VMEM((TOK_TILE, 1), jnp.float32),
        ],
    )(dist, thr, d0, d0.reshape(1, N), memory_slots, W4,
      b4.reshape(1, INPUT_DIM), ln2_g.reshape(1, INPUT_DIM),
      ln2_b.reshape(1, INPUT_DIM))

    return out.reshape(B, S, INPUT_DIM)
